# Initial kernel scaffold; baseline (speedup 1.0000x reference)
import jax
import jax.numpy as jnp
from jax.experimental import pallas as pl


def kernel(dec1_probs, dec2_probs, dec1_tgt, dec2_tgt, constrainer):
    raise NotImplementedError("write your pallas kernel here")


# speedup vs baseline: 1.7392x; 1.7392x over previous
import functools

import jax
import jax.numpy as jnp
from jax import lax
from jax.experimental import pallas as pl
from jax.experimental.pallas import tpu as pltpu
from jax.experimental.pallas import tpu_sc as plsc

_LN2 = 0.6931471805599453
_SQRT2 = 1.4142135623730951


def _vlog(x):
    ix = lax.bitcast_convert_type(x, jnp.int32)
    e = (ix >> 23) - 127
    mi = (ix & 0x007FFFFF) | 0x3F800000
    m = lax.bitcast_convert_type(mi, jnp.float32)
    big = m > _SQRT2
    m = jnp.where(big, m * 0.5, m)
    e = jnp.where(big, e + 1, e)
    t = (m - 1.0) / (m + 1.0)
    t2 = t * t
    p = t * (2.0 + t2 * (0.6666667 + t2 * (0.4 + t2 * (0.2857143 + t2 * 0.22222222))))
    return e.astype(jnp.float32) * _LN2 + p


_GATHER_DNUMS = lax.GatherDimensionNumbers(
    offset_dims=(), collapsed_slice_dims=(0,), start_index_map=(0,))


def _vperm(v, idx):
    return lax.gather(v, idx.reshape(16, 1), _GATHER_DNUMS, (1,),
                      mode=lax.GatherScatterMode.PROMISE_IN_BOUNDS)


def _vsum_all(v):
    iot = lax.broadcasted_iota(jnp.int32, (16,), 0)
    for sh in (8, 4, 2, 1):
        v = v + _vperm(v, iot ^ sh)
    return v


def _make_sc_kernel(N, V1, V2):
    NW = 16
    NPW = N // NW
    NB = NPW // 128
    NC = NPW // 16
    mesh = plsc.VectorSubcoreMesh(core_axis_name="c", subcore_axis_name="s")

    @functools.partial(
        pl.kernel,
        mesh=mesh,
        out_type=jax.ShapeDtypeStruct((16,), jnp.float32),
        scratch_types=[
            pltpu.VMEM((NPW,), jnp.int32),
            pltpu.VMEM((NPW,), jnp.int32),
            pltpu.VMEM((NB, 128), jnp.int32),
            pltpu.VMEM((NB, 128), jnp.int32),
            pltpu.VMEM((NB, 128), jnp.int32),
            pltpu.VMEM((NB, 128), jnp.float32),
            pltpu.VMEM((NB, 128), jnp.float32),
            pltpu.VMEM((NB, 128), jnp.float32),
            pltpu.VMEM((64,), jnp.float32),
            pltpu.VMEM((NW * 64,), jnp.float32),
            pltpu.VMEM((16,), jnp.float32),
            pltpu.VMEM_SHARED((NW * 64,), jnp.float32),
            pltpu.SemaphoreType.DMA,
        ],
    )
    def sc_kernel(d1, d2, t1, t2, cons, out,
                  t1_v, t2_v, fi1_v, fi2_v, fic_v, a_v, b_v, c_v,
                  part_v, all_v, out_v, shared, sem):
        cid = lax.axis_index("c")
        sid = lax.axis_index("s")
        base = sid * NPW

        @pl.when(cid == 0)
        def _work():
            pltpu.sync_copy(t1.at[pl.ds(base, NPW)], t1_v)
            pltpu.sync_copy(t2.at[pl.ds(base, NPW)], t2_v)
            iot = lax.broadcasted_iota(jnp.int32, (16,), 0)
            for i in range(NC):
                t1c = t1_v[pl.ds(i * 16, 16)]
                t2c = t2_v[pl.ds(i * 16, 16)]
                s1 = jnp.where(t1c == -100, 0, t1c)
                s2 = jnp.where(t2c == -100, 0, t2c)
                pos = base + i * 16 + iot
                r, o = (i * 16) // 128, (i * 16) % 128
                fi1_v[r, pl.ds(o, 16)] = pos * V1 + s1
                fi2_v[r, pl.ds(o, 16)] = pos * V2 + s2
                fic_v[r, pl.ds(o, 16)] = s1 * V2 + s2
            cps = []
            for j in range(NB):
                cps.append(pltpu.async_copy(d1.at[fi1_v.at[j]], a_v.at[j], sem))
                cps.append(pltpu.async_copy(d2.at[fi2_v.at[j]], b_v.at[j], sem))
                cps.append(pltpu.async_copy(cons.at[fic_v.at[j]], c_v.at[j], sem))
            for cp in cps:
                cp.wait()
            zero = jnp.zeros((16,), jnp.float32)
            s1v, s2v, c1v, c2v = zero, zero, zero, zero
            for i in range(NC):
                r, o = (i * 16) // 128, (i * 16) % 128
                t1c = t1_v[pl.ds(i * 16, 16)]
                t2c = t2_v[pl.ds(i * 16, 16)]
                av = a_v[r, pl.ds(o, 16)]
                bv = b_v[r, pl.ds(o, 16)]
                cv = c_v[r, pl.ds(o, 16)]
                cc = jnp.clip(cv, 0.0, 1.0)
                l1 = _vlog(av * cc)
                l2 = _vlog(bv * cc)
                m1 = t1c != -100
                m2 = t2c != -100
                s1v = s1v + jnp.where(m1, l1, 0.0)
                s2v = s2v + jnp.where(m2, l2, 0.0)
                c1v = c1v + jnp.where(m1, 1.0, 0.0)
                c2v = c2v + jnp.where(m2, 1.0, 0.0)
            part_v[pl.ds(0, 16)] = s1v
            part_v[pl.ds(16, 16)] = s2v
            part_v[pl.ds(32, 16)] = c1v
            part_v[pl.ds(48, 16)] = c2v
            pltpu.sync_copy(part_v, shared.at[pl.ds(sid * 64, 64)])

        plsc.subcore_barrier()

        @pl.when(jnp.logical_and(cid == 0, sid == 0))
        def _reduce():
            pltpu.sync_copy(shared, all_v)
            zero = jnp.zeros((16,), jnp.float32)
            acc = [zero, zero, zero, zero]
            for w in range(NW):
                for k in range(4):
                    acc[k] = acc[k] + all_v[pl.ds(w * 64 + k * 16, 16)]
            S1 = _vsum_all(acc[0])
            S2 = _vsum_all(acc[1])
            C1 = _vsum_all(acc[2])
            C2 = _vsum_all(acc[3])
            out_v[...] = -(S1 / C1) - (S2 / C2)
            pltpu.sync_copy(out_v, out)

    return sc_kernel


def kernel(dec1_probs, dec2_probs, dec1_tgt, dec2_tgt, constrainer):
    B, L, V1 = dec1_probs.shape
    V2 = dec2_probs.shape[-1]
    N = B * L
    d1 = dec1_probs.reshape(N * V1)
    d2 = dec2_probs.reshape(N * V2)
    t1 = dec1_tgt.reshape(N).astype(jnp.int32)
    t2 = dec2_tgt.reshape(N).astype(jnp.int32)
    cons = constrainer.reshape(V1 * V2)
    out = _make_sc_kernel(N, V1, V2)(d1, d2, t1, t2, cons)
    return out[0]
